# baseline (speedup 1.0000x reference)
import jax
import jax.numpy as jnp
from jax import lax
from jax.experimental import pallas as pl
from jax.experimental.pallas import tpu as pltpu

_N = 10000


def _chunk_sizes():
    return [7680, 2320]


def _mm_t(a, b):
    return lax.dot_general(a, b, (((1,), (1,)), ((), ())),
                           preferred_element_type=jnp.float32)


def _body(iseq_ref, lh_ref, nodes_ref, batch_ref, encW_ref, encb_ref,
          Wih_ref, Whh_ref, bih_ref, bhh_ref, projW_ref, projb_ref,
          compW_ref, compb_ref, outW_ref, outb_ref,
          out_ref, hid_ref, attn_ref,
          nbuf, logits_s, sems):
    H = 256
    bs = 8
    sizes = _chunk_sizes()

    copies = []
    off = 0
    for i, sz in enumerate(sizes):
        cp = pltpu.make_async_copy(
            nodes_ref.at[pl.ds(off, sz), :],
            nbuf.at[pl.ds(off, sz), :],
            sems.at[i],
        )
        cp.start()
        copies.append(cp)
        off += sz

    x = jnp.maximum(_mm_t(iseq_ref[0], encW_ref[...]) + encb_ref[...][None, :], 0.0)
    h = lh_ref[0]
    gx = _mm_t(x, Wih_ref[...]) + bih_ref[...][None, :]
    gh = _mm_t(h, Whh_ref[...]) + bhh_ref[...][None, :]
    r = jax.nn.sigmoid(gx[:, :H] + gh[:, :H])
    z = jax.nn.sigmoid(gx[:, H:2 * H] + gh[:, H:2 * H])
    n = jnp.tanh(gx[:, 2 * H:] + r * gh[:, 2 * H:])
    h_new = (1.0 - z) * n + z * h
    hid_ref[0] = h_new
    q = _mm_t(jnp.maximum(h_new, 0.0), projW_ref[...]) + projb_ref[...][None, :]
    qbf = q.astype(jnp.bfloat16)

    brow = batch_ref[...].reshape(1, _N)

    m = jnp.full((bs, 1), -1e30, jnp.float32)
    S = jnp.zeros((bs, bs), jnp.float32)
    cacc = jnp.zeros((bs, H), jnp.float32)

    off = 0
    for i, sz in enumerate(sizes):
        copies[i].wait()
        nbf = nbuf[pl.ds(off, sz), :].astype(jnp.bfloat16)
        logits = _mm_t(qbf, nbf)
        logits_s[:, pl.ds(off, sz)] = logits

        bmax = jnp.max(logits, axis=1, keepdims=True)
        m_new = jnp.maximum(m, bmax)
        alpha = jnp.exp(m - m_new)
        e = jnp.exp(logits - m_new)
        e_bf = e.astype(jnp.bfloat16)

        rowid = lax.broadcasted_iota(jnp.int32, (bs, sz), 0)
        mask_bf = (rowid == brow[:, off:off + sz]).astype(jnp.bfloat16)

        S_blk = _mm_t(e_bf, mask_bf)
        ctx = lax.dot_general(e_bf * mask_bf, nbf,
                              (((1,), (0,)), ((), ())),
                              preferred_element_type=jnp.float32)
        S = S * alpha + S_blk
        cacc = cacc * alpha + ctx
        m = m_new
        off += sz

    rid = lax.broadcasted_iota(jnp.int32, (bs, bs), 0)
    cid = lax.broadcasted_iota(jnp.int32, (bs, bs), 1)
    Sdiag = jnp.sum(jnp.where(rid == cid, S, 0.0), axis=1, keepdims=True)
    context = cacc / Sdiag
    concat = jnp.concatenate([q, context, x], axis=1)
    co = jnp.maximum(_mm_t(concat, compW_ref[...]) + compb_ref[...][None, :], 0.0)
    out_ref[...] = _mm_t(co, outW_ref[...]) + outb_ref[...][None, :]

    lg = logits_s[...]
    ex = jnp.exp(lg - m)
    Rinv = 1.0 / S
    Rg = jnp.zeros((bs, _N), jnp.float32)
    for s in range(bs):
        mask = brow == s
        Rg = Rg + jnp.where(mask, Rinv[:, s:s + 1], 0.0)
    attn_ref[...] = ex * Rg


def kernel(input_seq, last_hidden, nodes, batch, enc_W, enc_b, Wih, Whh,
           bih, bhh, proj_W, proj_b, comp_W, comp_b, out_W, out_b):
    n_nodes, H = nodes.shape
    bs = input_seq.shape[1]
    out_dim = out_W.shape[0]

    vmem = lambda a: pl.BlockSpec(memory_space=pltpu.MemorySpace.VMEM)
    hbm = pl.BlockSpec(memory_space=pltpu.MemorySpace.HBM)

    out, hid, attn_w = pl.pallas_call(
        _body,
        in_specs=[
            vmem(input_seq), vmem(last_hidden), hbm, vmem(batch),
            vmem(enc_W), vmem(enc_b), vmem(Wih), vmem(Whh), vmem(bih),
            vmem(bhh), vmem(proj_W), vmem(proj_b), vmem(comp_W),
            vmem(comp_b), vmem(out_W), vmem(out_b),
        ],
        out_specs=[vmem(None), vmem(None), vmem(None)],
        out_shape=[
            jax.ShapeDtypeStruct((bs, out_dim), jnp.float32),
            jax.ShapeDtypeStruct((1, bs, H), jnp.float32),
            jax.ShapeDtypeStruct((bs, n_nodes), jnp.float32),
        ],
        scratch_shapes=[
            pltpu.VMEM((n_nodes, H), jnp.float32),
            pltpu.VMEM((bs, n_nodes), jnp.float32),
            pltpu.SemaphoreType.DMA((len(_chunk_sizes()),)),
        ],
    )(input_seq, last_hidden, nodes, batch, enc_W, enc_b, Wih, Whh,
      bih, bhh, proj_W, proj_b, comp_W, comp_b, out_W, out_b)

    return out, hid, attn_w


# speedup vs baseline: 1.0974x; 1.0974x over previous
import jax
import jax.numpy as jnp
from jax import lax
from jax.experimental import pallas as pl
from jax.experimental.pallas import tpu as pltpu

_N = 10000


def _chunk_sizes():
    return [5120, 4880]


def _mm_t(a, b):
    return lax.dot_general(a, b, (((1,), (1,)), ((), ())),
                           preferred_element_type=jnp.float32)


def _body(iseq_ref, lh_ref, nodes_ref, batch_ref, encW_ref, encb_ref,
          Wih_ref, Whh_ref, bih_ref, bhh_ref, projW_ref, projb_ref,
          compW_ref, compb_ref, outW_ref, outb_ref,
          out_ref, hid_ref, attn_ref,
          nbuf, logits_s, sems):
    H = 256
    bs = 8
    sizes = _chunk_sizes()

    copies = []
    off = 0
    for i, sz in enumerate(sizes):
        cp = pltpu.make_async_copy(
            nodes_ref.at[pl.ds(off, sz), :],
            nbuf.at[pl.ds(off, sz), :],
            sems.at[i],
        )
        cp.start()
        copies.append(cp)
        off += sz

    x = jnp.maximum(_mm_t(iseq_ref[0], encW_ref[...]) + encb_ref[...][None, :], 0.0)
    h = lh_ref[0]
    gx = _mm_t(x, Wih_ref[...]) + bih_ref[...][None, :]
    gh = _mm_t(h, Whh_ref[...]) + bhh_ref[...][None, :]
    r = jax.nn.sigmoid(gx[:, :H] + gh[:, :H])
    z = jax.nn.sigmoid(gx[:, H:2 * H] + gh[:, H:2 * H])
    n = jnp.tanh(gx[:, 2 * H:] + r * gh[:, 2 * H:])
    h_new = (1.0 - z) * n + z * h
    hid_ref[0] = h_new
    q = _mm_t(jnp.maximum(h_new, 0.0), projW_ref[...]) + projb_ref[...][None, :]
    qbf = q.astype(jnp.bfloat16)

    brow = batch_ref[...].reshape(1, _N)

    m = jnp.full((bs, 1), -1e30, jnp.float32)
    S = jnp.zeros((bs, bs), jnp.float32)
    cacc = jnp.zeros((bs, H), jnp.float32)

    off = 0
    for i, sz in enumerate(sizes):
        copies[i].wait()
        nbf = nbuf[pl.ds(off, sz), :].astype(jnp.bfloat16)
        logits = _mm_t(qbf, nbf)
        logits_s[:, pl.ds(off, sz)] = logits

        bmax = jnp.max(logits, axis=1, keepdims=True)
        m_new = jnp.maximum(m, bmax)
        alpha = jnp.exp(m - m_new)
        e = jnp.exp(logits - m_new)
        e_bf = e.astype(jnp.bfloat16)

        rowid = lax.broadcasted_iota(jnp.int32, (bs, sz), 0)
        mask_bf = (rowid == brow[:, off:off + sz]).astype(jnp.bfloat16)

        S_blk = _mm_t(e_bf, mask_bf)
        ctx = lax.dot_general(e_bf * mask_bf, nbf,
                              (((1,), (0,)), ((), ())),
                              preferred_element_type=jnp.float32)
        S = S * alpha + S_blk
        cacc = cacc * alpha + ctx
        m = m_new
        off += sz

    rid = lax.broadcasted_iota(jnp.int32, (bs, bs), 0)
    cid = lax.broadcasted_iota(jnp.int32, (bs, bs), 1)
    Sdiag = jnp.sum(jnp.where(rid == cid, S, 0.0), axis=1, keepdims=True)
    context = cacc / Sdiag
    concat = jnp.concatenate([q, context, x], axis=1)
    co = jnp.maximum(_mm_t(concat, compW_ref[...]) + compb_ref[...][None, :], 0.0)
    out_ref[...] = _mm_t(co, outW_ref[...]) + outb_ref[...][None, :]

    lg = logits_s[...]
    ex = jnp.exp(lg - m)
    Rinv = 1.0 / S
    Rg = jnp.zeros((bs, _N), jnp.float32)
    for s in range(bs):
        mask = brow == s
        Rg = Rg + jnp.where(mask, Rinv[:, s:s + 1], 0.0)
    attn_ref[...] = ex * Rg


def kernel(input_seq, last_hidden, nodes, batch, enc_W, enc_b, Wih, Whh,
           bih, bhh, proj_W, proj_b, comp_W, comp_b, out_W, out_b):
    n_nodes, H = nodes.shape
    bs = input_seq.shape[1]
    out_dim = out_W.shape[0]

    vmem = lambda a: pl.BlockSpec(memory_space=pltpu.MemorySpace.VMEM)
    hbm = pl.BlockSpec(memory_space=pltpu.MemorySpace.HBM)

    out, hid, attn_w = pl.pallas_call(
        _body,
        in_specs=[
            vmem(input_seq), vmem(last_hidden), hbm, vmem(batch),
            vmem(enc_W), vmem(enc_b), vmem(Wih), vmem(Whh), vmem(bih),
            vmem(bhh), vmem(proj_W), vmem(proj_b), vmem(comp_W),
            vmem(comp_b), vmem(out_W), vmem(out_b),
        ],
        out_specs=[vmem(None), vmem(None), vmem(None)],
        out_shape=[
            jax.ShapeDtypeStruct((bs, out_dim), jnp.float32),
            jax.ShapeDtypeStruct((1, bs, H), jnp.float32),
            jax.ShapeDtypeStruct((bs, n_nodes), jnp.float32),
        ],
        scratch_shapes=[
            pltpu.VMEM((n_nodes, H), jnp.float32),
            pltpu.VMEM((bs, n_nodes), jnp.float32),
            pltpu.SemaphoreType.DMA((len(_chunk_sizes()),)),
        ],
    )(input_seq, last_hidden, nodes, batch, enc_W, enc_b, Wih, Whh,
      bih, bhh, proj_W, proj_b, comp_W, comp_b, out_W, out_b)

    return out, hid, attn_w
